# Initial kernel scaffold; baseline (speedup 1.0000x reference)
import functools

import jax
import jax.numpy as jnp
from jax import lax
from jax.experimental import pallas as pl
from jax.experimental.pallas import tpu as pltpu
from jax.experimental.pallas import tpu_sc as plsc

N = 10000
E = 320000
D = 128
L = 16

NC = 2
NS = 16
NW = NC * NS
CHUNK = 128
E2 = 2 * E
CPW = -(-E2 // (NW * CHUNK))
EPAD = NW * CPW * CHUNK
ACC_N = N + 16
TBL_N = N + 1 + L

_BM = 1000


def _mm_body(x_ref, w_ref, o_ref):
    o_ref[...] = jnp.dot(x_ref[...], w_ref[...],
                         preferred_element_type=jnp.float32)


def _matmul(x, w):
    return pl.pallas_call(
        _mm_body,
        grid=(N // _BM,),
        in_specs=[pl.BlockSpec((_BM, D), lambda i: (i, 0)),
                  pl.BlockSpec((D, D), lambda i: (0, 0))],
        out_specs=pl.BlockSpec((_BM, D), lambda i: (i, 0)),
        out_shape=jax.ShapeDtypeStruct((N, D), jnp.float32),
    )(x, w)


def _final_body(x_ref, wl_ref, b_ref, p0_ref, p1_ref, o_ref):
    xl = lax.dot_general(x_ref[...], wl_ref[...],
                         (((1,), (1,)), ((), ())),
                         preferred_element_type=jnp.float32)
    o_ref[...] = jnp.maximum(xl + b_ref[...] + p0_ref[...] + p1_ref[...], 0.0)


def _final(x, w_lin, b_lin, p0, p1):
    return pl.pallas_call(
        _final_body,
        grid=(N // _BM,),
        in_specs=[pl.BlockSpec((_BM, D), lambda i: (i, 0)),
                  pl.BlockSpec((D, D), lambda i: (0, 0)),
                  pl.BlockSpec((1, D), lambda i: (0, 0)),
                  pl.BlockSpec((_BM, D), lambda i: (i, 0)),
                  pl.BlockSpec((_BM, D), lambda i: (i, 0))],
        out_specs=pl.BlockSpec((_BM, D), lambda i: (i, 0)),
        out_shape=jax.ShapeDtypeStruct((N, D), jnp.float32),
    )(x, w_lin, b_lin.reshape(1, D), p0, p1)


_sc_mesh = plsc.VectorSubcoreMesh(core_axis_name="c", subcore_axis_name="s")


@functools.partial(
    pl.kernel,
    out_type=jax.ShapeDtypeStruct((NC, N, D), jnp.float32),
    mesh=_sc_mesh,
    scratch_types=[
        pltpu.VMEM((CPW, CHUNK), jnp.int32),
        pltpu.VMEM((CPW, CHUNK), jnp.int32),
        pltpu.VMEM((CPW, CHUNK), jnp.int32),
        pltpu.VMEM((CHUNK, D), jnp.float32),
        pltpu.VMEM((CHUNK, D), jnp.float32),
        pltpu.VMEM_SHARED((ACC_N, D), jnp.float32),
        pltpu.SemaphoreType.DMA,
        pltpu.SemaphoreType.DMA,
    ],
)
def _sc_scatter(table_hbm, src_hbm, bidx_hbm, dst_hbm, zeros_hbm, out_hbm,
                src_v, bidx_v, dst_v, rows_v, brows_v, acc, sem1, sem2):
    c = lax.axis_index("c")
    s = lax.axis_index("s")
    gwid = c * NS + s

    zrows = ACC_N // NS
    pltpu.sync_copy(zeros_hbm.at[pl.ds(s * zrows, zrows)],
                    acc.at[pl.ds(s * zrows, zrows)])

    pltpu.sync_copy(src_hbm.at[gwid], src_v)
    pltpu.sync_copy(bidx_hbm.at[gwid], bidx_v)
    pltpu.sync_copy(dst_hbm.at[gwid], dst_v)

    plsc.subcore_barrier()

    def body(i, carry):
        pltpu.async_copy(table_hbm.at[src_v.at[i]], rows_v, sem1).wait()
        pltpu.async_copy(table_hbm.at[bidx_v.at[i]], brows_v, sem2).wait()
        pltpu.sync_copy(rows_v, acc.at[dst_v.at[i]], add=True)
        pltpu.sync_copy(brows_v, acc.at[dst_v.at[i]], add=True)
        return carry

    lax.fori_loop(0, CPW, body, 0)

    plsc.subcore_barrier()

    orows = N // NS
    pltpu.sync_copy(acc.at[pl.ds(s * orows, orows)],
                    out_hbm.at[c, pl.ds(s * orows, orows)])


def kernel(x, edge_index, edge_label, W_dir, b_lab, W_lin, b_lin):
    xw = _matmul(x, W_dir)
    table = jnp.concatenate(
        [xw, jnp.zeros((1, D), jnp.float32), b_lab], axis=0)

    src = jnp.concatenate([edge_index[0], edge_index[1]])
    dst = jnp.concatenate([edge_index[1], edge_index[0]])
    lab = jnp.concatenate([edge_label, edge_label]) + jnp.int32(N + 1)

    pad = EPAD - E2
    padv = jnp.full((pad,), N, jnp.int32)
    src_full = jnp.concatenate([src, padv]).reshape(NW, CPW, CHUNK)
    dst_full = jnp.concatenate([dst, padv]).reshape(NW, CPW, CHUNK)
    bidx_full = jnp.concatenate([lab, padv]).reshape(NW, CPW, CHUNK)

    zeros = jnp.zeros((ACC_N, D), jnp.float32)

    partials = _sc_scatter(table, src_full, bidx_full, dst_full, zeros)
    return _final(x, W_lin, b_lin, partials[0], partials[1])


# speedup vs baseline: 3.8237x; 3.8237x over previous
import functools

import jax
import jax.numpy as jnp
from jax import lax
from jax.experimental import pallas as pl
from jax.experimental.pallas import tpu as pltpu
from jax.experimental.pallas import tpu_sc as plsc

N = 10000
E = 320000
D = 128
L = 16

NC = 2
NS = 16
NW = NC * NS
CHUNK = 128
K = 16
E2 = 2 * E
CPW = 160
NB = CPW // K
EPAD = NW * CPW * CHUNK
ACC_N = 10112
TBL_N = N + 1 + L

_BM = 1000


def _mm_body(x_ref, w_ref, o_ref):
    o_ref[...] = jnp.dot(x_ref[...], w_ref[...],
                         preferred_element_type=jnp.float32)


def _matmul(x, w):
    return pl.pallas_call(
        _mm_body,
        grid=(N // _BM,),
        in_specs=[pl.BlockSpec((_BM, D), lambda i: (i, 0)),
                  pl.BlockSpec((D, D), lambda i: (0, 0))],
        out_specs=pl.BlockSpec((_BM, D), lambda i: (i, 0)),
        out_shape=jax.ShapeDtypeStruct((N, D), jnp.float32),
    )(x, w)


def _final_body(x_ref, wl_ref, b_ref, p0_ref, p1_ref, o_ref):
    xl = lax.dot_general(x_ref[...], wl_ref[...],
                         (((1,), (1,)), ((), ())),
                         preferred_element_type=jnp.float32)
    o_ref[...] = jnp.maximum(xl + b_ref[...] + p0_ref[...] + p1_ref[...], 0.0)


def _final(x, w_lin, b_lin, p0, p1):
    return pl.pallas_call(
        _final_body,
        grid=(N // _BM,),
        in_specs=[pl.BlockSpec((_BM, D), lambda i: (i, 0)),
                  pl.BlockSpec((D, D), lambda i: (0, 0)),
                  pl.BlockSpec((1, D), lambda i: (0, 0)),
                  pl.BlockSpec((_BM, D), lambda i: (i, 0)),
                  pl.BlockSpec((_BM, D), lambda i: (i, 0))],
        out_specs=pl.BlockSpec((_BM, D), lambda i: (i, 0)),
        out_shape=jax.ShapeDtypeStruct((N, D), jnp.float32),
    )(x, w_lin, b_lin.reshape(1, D), p0, p1)


_sc_mesh = plsc.VectorSubcoreMesh(core_axis_name="c", subcore_axis_name="s")


@functools.partial(
    pl.kernel,
    out_type=jax.ShapeDtypeStruct((NC, ACC_N, D), jnp.float32),
    mesh=_sc_mesh,
    scratch_types=[
        pltpu.VMEM((K, CHUNK), jnp.int32),
        pltpu.VMEM((K, CHUNK), jnp.int32),
        pltpu.VMEM((CHUNK, D), jnp.float32),
        pltpu.VMEM((CHUNK, D), jnp.float32),
        pltpu.VMEM_SHARED((ACC_N, D), jnp.float32),
        pltpu.SemaphoreType.DMA,
        pltpu.SemaphoreType.DMA,
    ],
)
def _sc_scatter(table_hbm, src_hbm, bidx_hbm, dst_hbm, zeros_hbm, out_hbm,
                gidx_v, didx_v, rows_a, rows_b, acc, sem_a, sem_b):
    c = lax.axis_index("c")
    s = lax.axis_index("s")
    gwid = c * NS + s

    zrows = ACC_N // NS
    pltpu.sync_copy(zeros_hbm.at[pl.ds(s * zrows, zrows)],
                    acc.at[pl.ds(s * zrows, zrows)])
    plsc.subcore_barrier()

    bufs = (rows_a, rows_b)
    sems = (sem_a, sem_b)

    def run_pass(idx_hbm):
        def block_body(kb, carry):
            pltpu.sync_copy(idx_hbm.at[gwid, pl.ds(kb * K, K)], gidx_v)
            pltpu.sync_copy(dst_hbm.at[gwid, pl.ds(kb * K, K)], didx_v)
            desc = pltpu.async_copy(table_hbm.at[gidx_v.at[0]],
                                    bufs[0], sems[0])
            for j in range(K):
                if j + 1 < K:
                    ndesc = pltpu.async_copy(
                        table_hbm.at[gidx_v.at[j + 1]],
                        bufs[(j + 1) % 2], sems[(j + 1) % 2])
                desc.wait()
                pltpu.sync_copy(bufs[j % 2], acc.at[didx_v.at[j]], add=True)
                if j + 1 < K:
                    desc = ndesc
            return carry

        lax.fori_loop(0, NB, block_body, 0)

    run_pass(src_hbm)
    run_pass(bidx_hbm)

    plsc.subcore_barrier()

    pltpu.sync_copy(acc.at[pl.ds(s * zrows, zrows)],
                    out_hbm.at[c, pl.ds(s * zrows, zrows)])


def kernel(x, edge_index, edge_label, W_dir, b_lab, W_lin, b_lin):
    xw = _matmul(x, W_dir)
    table = jnp.concatenate(
        [xw, jnp.zeros((1, D), jnp.float32), b_lab], axis=0)

    src = jnp.concatenate([edge_index[0], edge_index[1]])
    dst = jnp.concatenate([edge_index[1], edge_index[0]])
    lab = jnp.concatenate([edge_label, edge_label]) + jnp.int32(N + 1)

    pad = EPAD - E2
    padv = jnp.full((pad,), N, jnp.int32)
    src_full = jnp.concatenate([src, padv]).reshape(NW, CPW, CHUNK)
    dst_full = jnp.concatenate([dst, padv]).reshape(NW, CPW, CHUNK)
    bidx_full = jnp.concatenate([lab, padv]).reshape(NW, CPW, CHUNK)

    zeros = jnp.zeros((ACC_N, D), jnp.float32)

    partials = _sc_scatter(table, src_full, bidx_full, dst_full, zeros)
    return _final(x, W_lin, b_lin, partials[0], partials[1])
